# Initial kernel scaffold; baseline (speedup 1.0000x reference)
import jax
import jax.numpy as jnp
from jax.experimental import pallas as pl


def kernel(style_E, style_S, translate_E, translate_S, neg_idx):
    raise NotImplementedError("write your pallas kernel here")


# speedup vs baseline: 9.2679x; 9.2679x over previous
import functools

import jax
import jax.numpy as jnp
from jax.experimental import pallas as pl
from jax.experimental.pallas import tpu as pltpu

HW = 4096
CCHUNK = 16


def _bitonic_sort_rows(x):
  n = x.shape[-1]
  lane = jax.lax.broadcasted_iota(jnp.int32, (1, n), 1)
  k = 2
  while k <= n:
    j = k // 2
    while j >= 1:
      is_lower = (lane & j) == 0
      keep_lo = is_lower == ((lane & k) == 0)
      d = jnp.roll(x, -j, axis=1)
      u = jnp.roll(x, j, axis=1)
      partner = jnp.where(is_lower, d, u)
      x = jnp.where(keep_lo, jnp.minimum(x, partner), jnp.maximum(x, partner))
      j //= 2
    k *= 2
  return x


def _main_kernel(style_ref, trans_ref, out_ref, *, batch):
  ci = pl.program_id(1)
  s = style_ref[0].reshape(batch * CCHUNK, HW)
  t = trans_ref[0].reshape(batch * CCHUNK, HW)

  dn = (((1,), (1,)), ((), ()))
  rows = jax.lax.broadcasted_iota(jnp.int32, (batch * CCHUNK,) * 2, 0)
  cols = jax.lax.broadcasted_iota(jnp.int32, (batch * CCHUNK,) * 2, 1)
  diag = (rows % CCHUNK == cols % CCHUNK).astype(jnp.float32)
  pr = jax.lax.broadcasted_iota(jnp.int32, (batch * CCHUNK, batch), 0)
  pc = jax.lax.broadcasted_iota(jnp.int32, (batch * CCHUNK, batch), 1)
  proj = (pr // CCHUNK == pc).astype(jnp.float32)

  def blocksum(g):
    gp = jax.lax.dot_general(proj, g * diag, (((0,), (0,)), ((), ())),
                             preferred_element_type=jnp.float32)
    return jax.lax.dot_general(gp, proj, (((1,), (0,)), ((), ())),
                               preferred_element_type=jnp.float32)

  ss_s = blocksum(jax.lax.dot_general(s, s, dn, preferred_element_type=jnp.float32))
  ss_t = blocksum(jax.lax.dot_general(t, t, dn, preferred_element_type=jnp.float32))

  s_sorted = _bitonic_sort_rows(s)
  t_sorted = _bitonic_sort_rows(t)
  g = blocksum(jax.lax.dot_general(s_sorted, t_sorted, dn,
                                   preferred_element_type=jnp.float32))

  part = jnp.concatenate(
      [g, ss_s, ss_t, jnp.zeros((batch, batch), jnp.float32)], axis=0)
  part = jnp.pad(part, ((0, 0), (0, 128 - batch)))

  @pl.when(ci == 0)
  def _():
    out_ref[...] = part[None]

  @pl.when(ci != 0)
  def _():
    out_ref[...] += part[None]


def _main_call(style, trans):
  groups, batch, chans, _ = style.shape
  assert chans % CCHUNK == 0
  kern = functools.partial(_main_kernel, batch=batch)
  return pl.pallas_call(
      kern,
      grid=(groups, chans // CCHUNK),
      in_specs=[
          pl.BlockSpec((1, batch, CCHUNK, HW), lambda g, c: (g, 0, c, 0)),
          pl.BlockSpec((1, batch, CCHUNK, HW), lambda g, c: (g, 0, c, 0)),
      ],
      out_specs=pl.BlockSpec((1, 4 * batch, 128), lambda g, c: (g, 0, 0)),
      out_shape=jax.ShapeDtypeStruct((groups, 4 * batch, 128), jnp.float32),
      compiler_params=pltpu.CompilerParams(
          dimension_semantics=("arbitrary", "arbitrary")),
  )(style, trans)


def _epilogue_kernel(p_ref, neg_ref, out_ref, *, batch, n_neg, scale):
  pt = jnp.sum(p_ref[...], axis=0)
  g = pt[0:batch, 0:batch]
  ss_s = pt[batch:2 * batch, 0:batch]
  ss_t = pt[2 * batch:3 * batch, 0:batch]

  ri = jax.lax.broadcasted_iota(jnp.int32, (batch, batch), 0)
  ci = jax.lax.broadcasted_iota(jnp.int32, (batch, batch), 1)
  eye = (ri == ci).astype(jnp.float32)
  g_diag = jnp.sum(g * eye, axis=1, keepdims=True)
  ss_s_diag = jnp.sum(ss_s * eye, axis=1, keepdims=True)
  ss_t_diag = jnp.sum(ss_t * eye, axis=1, keepdims=True)
  poss = (ss_s_diag + ss_t_diag - 2.0 * g_diag) * scale

  ri1 = jax.lax.broadcasted_iota(jnp.int32, (batch, 1), 0)
  loss = jnp.float32(0.0)
  for b in range(batch):
    neg = jnp.float32(0.0)
    ss_t_b = jnp.sum(ss_t_diag * (ri1 == b))
    for j in range(n_neg):
      nb = neg_ref[b, j]
      ss_s_nb = jnp.sum(ss_s_diag * (ri1 == nb))
      g_nb_b = jnp.sum(g * (ri == nb) * (ci == b))
      neg = neg + (ss_s_nb + ss_t_b - 2.0 * g_nb_b) * scale
    poss_b = jnp.sum(poss * (ri1 == b))
    loss = loss + poss_b / neg
  out_ref[0, 0] = loss


def _epilogue_call(p, neg_idx, chans):
  groups, rows, _ = p.shape
  batch = rows // 4
  n_neg = neg_idx.shape[1]
  kern = functools.partial(_epilogue_kernel, batch=batch, n_neg=n_neg,
                           scale=1.0 / (chans * HW))
  return pl.pallas_call(
      kern,
      in_specs=[
          pl.BlockSpec(memory_space=pltpu.MemorySpace.VMEM),
          pl.BlockSpec(memory_space=pltpu.MemorySpace.SMEM),
      ],
      out_specs=pl.BlockSpec(memory_space=pltpu.MemorySpace.SMEM),
      out_shape=jax.ShapeDtypeStruct((1, 1), jnp.float32),
  )(p, neg_idx)


@jax.jit
def kernel(style_E, style_S, translate_E, translate_S, neg_idx):
  le, _, batch, chans, w, h = style_E.shape
  ls = style_S.shape[0]
  pe = _main_call(style_E.reshape(le * 2, batch, chans, w * h),
                  translate_E.reshape(le * 2, batch, chans, w * h))
  ps = _main_call(style_S.reshape(ls * 2, batch, chans, w * h),
                  translate_S.reshape(ls * 2, batch, chans, w * h))
  p = jnp.concatenate([pe, ps], axis=0)
  loss = _epilogue_call(p, neg_idx, chans)
  return loss[0, 0]


# speedup vs baseline: 9.8495x; 1.0628x over previous
import functools

import jax
import jax.numpy as jnp
from jax.experimental import pallas as pl
from jax.experimental.pallas import tpu as pltpu

HW = 4096
CCHUNK = 32


def _bitonic_sort_rows(x):
  n = x.shape[-1]
  lane = jax.lax.broadcasted_iota(jnp.int32, (1, n), 1)
  k = 2
  while k <= n:
    j = k // 2
    while j >= 1:
      is_lower = (lane & j) == 0
      keep_lo = is_lower == ((lane & k) == 0)
      d = jnp.roll(x, -j, axis=1)
      u = jnp.roll(x, j, axis=1)
      partner = jnp.where(is_lower, d, u)
      x = jnp.where(keep_lo, jnp.minimum(x, partner), jnp.maximum(x, partner))
      j //= 2
    k *= 2
  return x


def _main_kernel(style_ref, trans_ref, out_ref, *, batch):
  ci = pl.program_id(1)
  s = style_ref[0].reshape(batch * CCHUNK, HW)
  t = trans_ref[0].reshape(batch * CCHUNK, HW)

  dn = (((1,), (1,)), ((), ()))
  rows = jax.lax.broadcasted_iota(jnp.int32, (batch * CCHUNK,) * 2, 0)
  cols = jax.lax.broadcasted_iota(jnp.int32, (batch * CCHUNK,) * 2, 1)
  diag = (rows % CCHUNK == cols % CCHUNK).astype(jnp.float32)
  pr = jax.lax.broadcasted_iota(jnp.int32, (batch * CCHUNK, batch), 0)
  pc = jax.lax.broadcasted_iota(jnp.int32, (batch * CCHUNK, batch), 1)
  proj = (pr // CCHUNK == pc).astype(jnp.float32)

  def blocksum(g):
    gp = jax.lax.dot_general(proj, g * diag, (((0,), (0,)), ((), ())),
                             preferred_element_type=jnp.float32)
    return jax.lax.dot_general(gp, proj, (((1,), (0,)), ((), ())),
                               preferred_element_type=jnp.float32)

  ss_s = blocksum(jax.lax.dot_general(s, s, dn, preferred_element_type=jnp.float32))
  ss_t = blocksum(jax.lax.dot_general(t, t, dn, preferred_element_type=jnp.float32))

  st_sorted = _bitonic_sort_rows(jnp.concatenate([s, t], axis=0))
  s_sorted = st_sorted[:batch * CCHUNK]
  t_sorted = st_sorted[batch * CCHUNK:]
  g = blocksum(jax.lax.dot_general(s_sorted, t_sorted, dn,
                                   preferred_element_type=jnp.float32))

  part = jnp.concatenate(
      [g, ss_s, ss_t, jnp.zeros((batch, batch), jnp.float32)], axis=0)
  part = jnp.pad(part, ((0, 0), (0, 128 - batch)))

  @pl.when(ci == 0)
  def _():
    out_ref[...] = part[None]

  @pl.when(ci != 0)
  def _():
    out_ref[...] += part[None]


def _main_call(style, trans):
  groups, batch, chans, _ = style.shape
  assert chans % CCHUNK == 0
  kern = functools.partial(_main_kernel, batch=batch)
  return pl.pallas_call(
      kern,
      grid=(groups, chans // CCHUNK),
      in_specs=[
          pl.BlockSpec((1, batch, CCHUNK, HW), lambda g, c: (g, 0, c, 0)),
          pl.BlockSpec((1, batch, CCHUNK, HW), lambda g, c: (g, 0, c, 0)),
      ],
      out_specs=pl.BlockSpec((1, 4 * batch, 128), lambda g, c: (g, 0, 0)),
      out_shape=jax.ShapeDtypeStruct((groups, 4 * batch, 128), jnp.float32),
      compiler_params=pltpu.CompilerParams(
          dimension_semantics=("arbitrary", "arbitrary")),
  )(style, trans)


def _epilogue_kernel(p_ref, neg_ref, out_ref, *, batch, n_neg, scale):
  pt = jnp.sum(p_ref[...], axis=0)
  g = pt[0:batch, 0:batch]
  ss_s = pt[batch:2 * batch, 0:batch]
  ss_t = pt[2 * batch:3 * batch, 0:batch]

  ri = jax.lax.broadcasted_iota(jnp.int32, (batch, batch), 0)
  ci = jax.lax.broadcasted_iota(jnp.int32, (batch, batch), 1)
  eye = (ri == ci).astype(jnp.float32)
  g_diag = jnp.sum(g * eye, axis=1, keepdims=True)
  ss_s_diag = jnp.sum(ss_s * eye, axis=1, keepdims=True)
  ss_t_diag = jnp.sum(ss_t * eye, axis=1, keepdims=True)
  poss = (ss_s_diag + ss_t_diag - 2.0 * g_diag) * scale

  ri1 = jax.lax.broadcasted_iota(jnp.int32, (batch, 1), 0)
  loss = jnp.float32(0.0)
  for b in range(batch):
    neg = jnp.float32(0.0)
    ss_t_b = jnp.sum(ss_t_diag * (ri1 == b))
    for j in range(n_neg):
      nb = neg_ref[b, j]
      ss_s_nb = jnp.sum(ss_s_diag * (ri1 == nb))
      g_nb_b = jnp.sum(g * (ri == nb) * (ci == b))
      neg = neg + (ss_s_nb + ss_t_b - 2.0 * g_nb_b) * scale
    poss_b = jnp.sum(poss * (ri1 == b))
    loss = loss + poss_b / neg
  out_ref[0, 0] = loss


def _epilogue_call(p, neg_idx, chans):
  groups, rows, _ = p.shape
  batch = rows // 4
  n_neg = neg_idx.shape[1]
  kern = functools.partial(_epilogue_kernel, batch=batch, n_neg=n_neg,
                           scale=1.0 / (chans * HW))
  return pl.pallas_call(
      kern,
      in_specs=[
          pl.BlockSpec(memory_space=pltpu.MemorySpace.VMEM),
          pl.BlockSpec(memory_space=pltpu.MemorySpace.SMEM),
      ],
      out_specs=pl.BlockSpec(memory_space=pltpu.MemorySpace.SMEM),
      out_shape=jax.ShapeDtypeStruct((1, 1), jnp.float32),
  )(p, neg_idx)


@jax.jit
def kernel(style_E, style_S, translate_E, translate_S, neg_idx):
  le, _, batch, chans, w, h = style_E.shape
  ls = style_S.shape[0]
  pe = _main_call(style_E.reshape(le * 2, batch, chans, w * h),
                  translate_E.reshape(le * 2, batch, chans, w * h))
  ps = _main_call(style_S.reshape(ls * 2, batch, chans, w * h),
                  translate_S.reshape(ls * 2, batch, chans, w * h))
  p = jnp.concatenate([pe, ps], axis=0)
  loss = _epilogue_call(p, neg_idx, chans)
  return loss[0, 0]
